# baseline (speedup 1.0000x reference)
import functools

import jax
import jax.numpy as jnp
from jax import lax
from jax.experimental import pallas as pl
from jax.experimental.pallas import tpu as pltpu
from jax.experimental.pallas import tpu_sc as plsc

NUM_MASHUP = 8000
NUM_API = 2000
WPR = NUM_API // 2
WORDS = NUM_MASHUP * WPR
FEAT = 128
CH = 768
BETA = 0.5
F32 = jnp.float32
BF16 = jnp.bfloat16

SC_NC = 2
SC_NS = 16
SC_NW = SC_NC * SC_NS
LANES = 16
WSIZE = 32768

EVEN_ONE = 0x3F80
ODD_ONE = 0x3F800000


def _cdiv(a, b):
    return (a + b - 1) // b


def _build_adjacency_words(src_half, dst_half, e2):
    nper = _cdiv(_cdiv(e2, LANES), SC_NW) * LANES
    ld = nper + 64
    gpmax = ld // LANES

    mesh = plsc.VectorSubcoreMesh(
        core_axis_name="c", subcore_axis_name="s",
        num_cores=SC_NC, num_subcores=SC_NS,
    )

    @functools.partial(
        pl.kernel,
        mesh=mesh,
        out_type=jax.ShapeDtypeStruct((WORDS,), jnp.int32),
        scratch_types=[
            pltpu.VMEM((ld,), jnp.int32),
            pltpu.VMEM((ld,), jnp.int32),
            pltpu.VMEM((WSIZE,), jnp.int32),
        ],
        compiler_params=pltpu.CompilerParams(needs_layout_passes=False),
    )
    def build(src_hbm, dst_hbm, zero_hbm, a_out, src_v, dst_v, win_v):
        wid = lax.axis_index("s") * SC_NC + lax.axis_index("c")
        base = wid * nper
        off0 = pl.multiple_of(jnp.maximum(base - 32, 0), 32)
        pltpu.sync_copy(src_hbm.at[pl.ds(off0, ld)], src_v)
        pltpu.sync_copy(dst_hbm.at[pl.ds(off0, ld)], dst_v)
        pltpu.sync_copy(zero_hbm, win_v)

        iota = lax.iota(jnp.int32, LANES)
        vals_even = jnp.full((LANES,), EVEN_ONE, jnp.int32)
        vals_odd = jnp.full((LANES,), ODD_ONE, jnp.int32)

        def word_at(pos):
            pos = pl.multiple_of(pos, LANES)
            s = src_v[pl.ds(pos, LANES)]
            d = dst_v[pl.ds(pos, LANES)]
            return s * WPR + ((d - NUM_MASHUP) >> 1)

        pos_b = base - off0
        wstart = jnp.where(wid == 0, 0, word_at(pos_b)[0] & ~15)
        pos_n = base + nper - off0
        wend_all = jnp.where(
            wid == SC_NW - 1, WORDS, word_at(pos_n)[0] & ~15
        )

        nsteps = gpmax + (wend_all - wstart) // WSIZE + 3

        def step(t, state):
            wb, gp = state
            wb = pl.multiple_of(wb, LANES)
            wlen = jnp.minimum(WSIZE, wend_all - wb)
            wend = wb + wlen
            live = wb < wend_all

            gpc = jnp.minimum(gp, gpmax - 1)
            pos = pl.multiple_of(gpc * LANES, LANES)
            s = src_v[pl.ds(pos, LANES)]
            d = dst_v[pl.ds(pos, LANES)]
            a = d - NUM_MASHUP
            wi = s * WPR + (a >> 1)
            gid = off0 + pos + iota
            wi = jnp.where(gid < e2, wi, WORDS)
            have_grp = (gp < gpmax) & live
            wi = wi + (1 - have_grp.astype(jnp.int32)) * WORDS
            inw = (wi >= wb) & (wi < wend)
            li = jnp.where(inw, wi - wb, 0)
            even = (a & 1) == 0
            plsc.addupdate_scatter(win_v, [li], vals_even, mask=inw & even)
            plsc.addupdate_scatter(
                win_v, [li], vals_odd, mask=inw & jnp.logical_not(even)
            )
            consumed = plsc.all_reduce_population_count(wi < wend)[0]
            group_done = (consumed == LANES) & have_grp
            close = jnp.logical_not(group_done) & live

            for sz in (32768, 16384, 8192, 4096, 2048, 1024, 512, 256, 128,
                       64, 32, 16):
                if sz > WSIZE:
                    continue
                pre = pl.multiple_of(wlen & ~(2 * sz - 1), LANES)

                @pl.when(close & ((wlen & sz) != 0))
                def _(sz=sz, pre=pre):
                    pltpu.sync_copy(
                        win_v.at[pl.ds(pre, sz)],
                        a_out.at[pl.ds(pl.multiple_of(wb + pre, LANES), sz)],
                    )

            @pl.when(close)
            def _():
                pltpu.sync_copy(zero_hbm, win_v)

            wb2 = jnp.where(close, wb + wlen, wb)
            gp2 = jnp.where(group_done, gp + 1, gp)
            return wb2, gp2

        lax.fori_loop(0, nsteps, step, (wstart, jnp.int32(0)))

    zero_win = jnp.zeros((WSIZE,), jnp.int32)
    return build(src_half, dst_half, zero_win)


def _attention(x, dom, w_sde, b_sde, w_val, b_val, w_key, b_key):
    def body(x_r, dom_r, wsde_r, bsde_r, wval_r, bval_r, wkey_r, bkey_r, z_r):
        v_mi = jax.nn.sigmoid(
            jnp.dot(x_r[...], wsde_r[...], preferred_element_type=F32) + bsde_r[...]
        )
        v_val = jax.nn.sigmoid(
            jnp.dot(dom_r[...], wval_r[...], preferred_element_type=F32) + bval_r[...]
        )
        v_key = jax.nn.sigmoid(
            jnp.dot(dom_r[...], wkey_r[...], preferred_element_type=F32) + bkey_r[...]
        )
        al = lax.dot_general(
            v_mi, v_key, (((1,), (1,)), ((), ())), preferred_element_type=F32
        )
        alpha = al / jnp.sum(al, axis=1, keepdims=True)
        s_m = jnp.dot(alpha, v_val, preferred_element_type=F32)
        z_r[...] = (1.0 - BETA) * s_m + BETA * v_mi

    b = x.shape[0]
    return pl.pallas_call(
        body, out_shape=jax.ShapeDtypeStruct((b, FEAT), F32)
    )(x, dom, w_sde, b_sde, w_val, b_val, w_key, b_key)


def _proj(inp, w, b2d, blk):
    n = inp.shape[0]
    g = n // blk

    def body(i_r, w_r, b_r, o_r):
        o_r[...] = jax.nn.sigmoid(
            jnp.dot(i_r[...], w_r[...], preferred_element_type=F32) + b_r[...]
        )

    return pl.pallas_call(
        body,
        grid=(g,),
        in_specs=[
            pl.BlockSpec((blk, CH), lambda i: (i, 0)),
            pl.BlockSpec((CH, FEAT), lambda i: (0, 0)),
            pl.BlockSpec((1, FEAT), lambda i: (0, 0)),
        ],
        out_specs=pl.BlockSpec((blk, FEAT), lambda i: (i, 0)),
        out_shape=jax.ShapeDtypeStruct((n, FEAT), F32),
    )(inp, w, b2d)


RBLK = 1000
NSTEP = NUM_MASHUP // RBLK


def _rsqrtz(x):
    return jnp.where(x > 0, lax.rsqrt(x), 0.0)


def _gcn1(a16, v_m, v_s):
    def body(a_r, vm_r, vs_r, w_r, dega_r, accu_r, accda_r):
        i = pl.program_id(0)
        ab = a_r[...]
        degm = lax.dot_general(
            ab, jnp.ones((NUM_API, 8), BF16), (((1,), (0,)), ((), ())),
            preferred_element_type=F32,
        )[:, 0:1]
        dism = _rsqrtz(degm)
        contrib_u = lax.dot_general(
            ab, (dism * vm_r[...]).astype(BF16), (((0,), (0,)), ((), ())),
            preferred_element_type=F32,
        )
        contrib_da = lax.dot_general(
            ab, jnp.ones((RBLK, 8), BF16), (((0,), (0,)), ((), ())),
            preferred_element_type=F32,
        )

        @pl.when(i == 0)
        def _():
            accu_r[...] = contrib_u
            accda_r[...] = contrib_da

        @pl.when(i > 0)
        def _():
            accu_r[...] += contrib_u
            accda_r[...] += contrib_da

        @pl.when(i == NSTEP - 1)
        def _():
            dega = accda_r[...]
            disa = _rsqrtz(dega[:, 0:1])
            w_r[...] = vs_r[...] + disa * accu_r[...]
            dega_r[...] = dega

    return pl.pallas_call(
        body,
        grid=(NSTEP,),
        in_specs=[
            pl.BlockSpec((RBLK, NUM_API), lambda i: (i, 0)),
            pl.BlockSpec((RBLK, FEAT), lambda i: (i, 0)),
            pl.BlockSpec((NUM_API, FEAT), lambda i: (0, 0)),
        ],
        out_specs=[
            pl.BlockSpec((NUM_API, FEAT), lambda i: (0, 0)),
            pl.BlockSpec((NUM_API, 8), lambda i: (0, 0)),
        ],
        out_shape=[
            jax.ShapeDtypeStruct((NUM_API, FEAT), F32),
            jax.ShapeDtypeStruct((NUM_API, 8), F32),
        ],
        scratch_shapes=[
            pltpu.VMEM((NUM_API, FEAT), F32),
            pltpu.VMEM((NUM_API, 8), F32),
        ],
    )(a16, v_m, v_s)


def _gcn2(a16, w, dega, z_m):
    b = z_m.shape[0]

    def body(a_r, w_r, dega_r, zm_r, pred_r, acc2_r):
        i = pl.program_id(0)
        ab = a_r[...]
        disa = _rsqrtz(dega_r[:, 0:1])
        wa = (disa * w_r[...]).astype(BF16)
        traw = lax.dot_general(
            ab, wa, (((1,), (0,)), ((), ())), preferred_element_type=F32
        )
        degm = lax.dot_general(
            ab, jnp.ones((NUM_API, 8), BF16), (((1,), (0,)), ((), ())),
            preferred_element_type=F32,
        )[:, 0:1]
        dm2 = jnp.where(degm > 0, 1.0 / degm, 0.0)
        contrib = lax.dot_general(
            ab, (dm2 * traw).astype(BF16), (((0,), (0,)), ((), ())),
            preferred_element_type=F32,
        )

        @pl.when(i == 0)
        def _():
            acc2_r[...] = contrib

        @pl.when(i > 0)
        def _():
            acc2_r[...] += contrib

        @pl.when(i == NSTEP - 1)
        def _():
            out_s = 0.25 * (w_r[...] + disa * acc2_r[...])
            pred_r[...] = lax.dot_general(
                zm_r[...], out_s, (((1,), (1,)), ((), ())),
                preferred_element_type=F32,
            )

    return pl.pallas_call(
        body,
        grid=(NSTEP,),
        in_specs=[
            pl.BlockSpec((RBLK, NUM_API), lambda i: (i, 0)),
            pl.BlockSpec((NUM_API, FEAT), lambda i: (0, 0)),
            pl.BlockSpec((NUM_API, 8), lambda i: (0, 0)),
            pl.BlockSpec((b, FEAT), lambda i: (0, 0)),
        ],
        out_specs=pl.BlockSpec((b, NUM_API), lambda i: (0, 0)),
        out_shape=jax.ShapeDtypeStruct((b, NUM_API), F32),
        scratch_shapes=[pltpu.VMEM((NUM_API, FEAT), F32)],
    )(a16, w, dega, z_m)


def kernel(x, mashup_embed, api_embed, domain_embed, edge_index,
           W_sde, b_sde, W_val, b_val, W_key, b_key, W_sie, b_sie):
    e = edge_index.shape[1]
    e2 = e // 2
    src_half = edge_index[0]
    dst_half = edge_index[1]

    a_words = _build_adjacency_words(src_half, dst_half, e2)
    a16 = lax.bitcast_convert_type(
        a_words.reshape(NUM_MASHUP, WPR), BF16
    ).reshape(NUM_MASHUP, NUM_API)

    b_sde2 = b_sde.reshape(1, FEAT)
    b_val2 = b_val.reshape(1, FEAT)
    b_key2 = b_key.reshape(1, FEAT)
    b_sie2 = b_sie.reshape(1, FEAT)

    z_m = _attention(x, domain_embed, W_sde, b_sde2, W_val, b_val2, W_key, b_key2)
    v_m = _proj(mashup_embed, W_sde, b_sde2, 1000)
    v_s = _proj(api_embed, W_sie, b_sie2, 1000)

    w, dega = _gcn1(a16, v_m, v_s)
    pred = _gcn2(a16, w, dega, z_m)
    return pred


# speedup vs baseline: 3.9874x; 3.9874x over previous
import functools

import jax
import jax.numpy as jnp
from jax import lax
from jax.experimental import pallas as pl
from jax.experimental.pallas import tpu as pltpu
from jax.experimental.pallas import tpu_sc as plsc

NUM_MASHUP = 8000
NUM_API = 2000
WPR = NUM_API // 2
WORDS = NUM_MASHUP * WPR
FEAT = 128
CH = 768
BETA = 0.5
F32 = jnp.float32
BF16 = jnp.bfloat16

SC_NC = 2
SC_NS = 16
SC_NW = SC_NC * SC_NS
LANES = 16
WSIZE = 32768

EVEN_ONE = 0x3F80
ODD_ONE = 0x3F800000


def _cdiv(a, b):
    return (a + b - 1) // b


BACKS = 4096
WROWS = 32
WWORDS = WROWS * WPR


def _build_adjacency_words(src_half, dst_half, e2):
    nper = _cdiv(_cdiv(e2, LANES), SC_NW) * LANES
    ld = nper + BACKS + 64
    gpmax = ld // LANES

    mesh = plsc.VectorSubcoreMesh(
        core_axis_name="c", subcore_axis_name="s",
        num_cores=SC_NC, num_subcores=SC_NS,
    )

    @functools.partial(
        pl.kernel,
        mesh=mesh,
        out_type=jax.ShapeDtypeStruct((WORDS,), jnp.int32),
        scratch_types=[
            pltpu.VMEM((ld,), jnp.int32),
            pltpu.VMEM((ld,), jnp.int32),
            pltpu.VMEM((2 * WWORDS,), jnp.int32),
        ],
        compiler_params=pltpu.CompilerParams(needs_layout_passes=False),
    )
    def build(src_hbm, dst_hbm, zero_hbm, a_out, src_v, dst_v, win_v):
        wid = lax.axis_index("s") * SC_NC + lax.axis_index("c")
        base = wid * nper
        off0 = pl.multiple_of(jnp.maximum(base - BACKS, 0), 32)
        pltpu.sync_copy(src_hbm.at[pl.ds(off0, ld)], src_v)
        pltpu.sync_copy(dst_hbm.at[pl.ds(off0, ld)], dst_v)
        pltpu.sync_copy(zero_hbm, win_v.at[pl.ds(0, WWORDS)])
        pltpu.sync_copy(zero_hbm, win_v.at[pl.ds(WWORDS, WWORDS)])

        iota = lax.iota(jnp.int32, LANES)
        vals_lo = jnp.full((LANES,), 1, jnp.int32)
        vals_hi = jnp.full((LANES,), 1 << 16, jnp.int32)

        def row_at(pos):
            pos = pl.multiple_of(pos, LANES)
            return src_v[pl.ds(pos, LANES)]

        rstart = jnp.where(wid == 0, 0, row_at(base - off0)[0] & ~1)
        rend = jnp.where(
            wid == SC_NW - 1, NUM_MASHUP, row_at(base + nper - off0)[0] & ~1
        )

        nsteps = gpmax + (rend - rstart) // WROWS + 3

        def step(t, state):
            rb, gp, par = state
            pbase = par * WWORDS
            rlen = jnp.minimum(WROWS, rend - rb)
            rwend = rb + rlen
            live = rb < rend

            gpc = jnp.minimum(gp, gpmax - 1)
            pos = pl.multiple_of(gpc * LANES, LANES)
            s = src_v[pl.ds(pos, LANES)]
            d = dst_v[pl.ds(pos, LANES)]
            a = d - NUM_MASHUP
            gid = off0 + pos + iota
            have_grp = (gp < gpmax) & live
            rr = jnp.where(gid < e2, s, NUM_MASHUP)
            rr = rr + (1 - have_grp.astype(jnp.int32)) * NUM_MASHUP
            inw = (rr >= rb) & (rr < rwend)
            wc = jnp.where(a < WPR, a, a - WPR)
            li = jnp.where(inw, (rr - rb) * WPR + wc, 0) + pbase
            lo = inw & (a < WPR)
            hi = inw & jnp.logical_not(a < WPR)
            plsc.addupdate_scatter(win_v, [li], vals_lo, mask=lo)
            plsc.addupdate_scatter(win_v, [li], vals_hi, mask=hi)
            group_done = rr[LANES - 1] < rwend
            close = jnp.logical_not(group_done) & live

            @pl.when(close)
            def _():
                pl.delay(200)
                for nr in (32, 16, 8, 4, 2):
                    pre = rlen & ~(2 * nr - 1)

                    @pl.when((rlen & nr) != 0)
                    def _(nr=nr, pre=pre):
                        pltpu.sync_copy(
                            win_v.at[pl.ds(
                                pl.multiple_of(pbase + pre * WPR, 8),
                                nr * WPR)],
                            a_out.at[pl.ds(
                                pl.multiple_of((rb + pre) * WPR, 8), nr * WPR)],
                        )
                pltpu.sync_copy(
                    zero_hbm,
                    win_v.at[pl.ds(pl.multiple_of(pbase, 8), WWORDS)],
                )

            rb2 = jnp.where(close, rb + rlen, rb)
            gp2 = jnp.where(group_done, gp + 1, gp)
            par2 = jnp.where(close, 1 - par, par)
            return rb2, gp2, par2

        lax.fori_loop(0, nsteps, step, (rstart, jnp.int32(0), jnp.int32(0)))

    zero_win = jnp.zeros((WWORDS,), jnp.int32)
    return build(src_half, dst_half, zero_win)


def _attention(x, dom, w_sde, b_sde, w_val, b_val, w_key, b_key):
    def body(x_r, dom_r, wsde_r, bsde_r, wval_r, bval_r, wkey_r, bkey_r, z_r):
        v_mi = jax.nn.sigmoid(
            jnp.dot(x_r[...], wsde_r[...], preferred_element_type=F32) + bsde_r[...]
        )
        v_val = jax.nn.sigmoid(
            jnp.dot(dom_r[...], wval_r[...], preferred_element_type=F32) + bval_r[...]
        )
        v_key = jax.nn.sigmoid(
            jnp.dot(dom_r[...], wkey_r[...], preferred_element_type=F32) + bkey_r[...]
        )
        al = lax.dot_general(
            v_mi, v_key, (((1,), (1,)), ((), ())), preferred_element_type=F32
        )
        alpha = al / jnp.sum(al, axis=1, keepdims=True)
        s_m = jnp.dot(alpha, v_val, preferred_element_type=F32)
        z_r[...] = (1.0 - BETA) * s_m + BETA * v_mi

    b = x.shape[0]
    return pl.pallas_call(
        body, out_shape=jax.ShapeDtypeStruct((b, FEAT), F32)
    )(x, dom, w_sde, b_sde, w_val, b_val, w_key, b_key)


def _proj(inp, w, b2d, blk):
    n = inp.shape[0]
    g = n // blk

    def body(i_r, w_r, b_r, o_r):
        o_r[...] = jax.nn.sigmoid(
            jnp.dot(i_r[...], w_r[...], preferred_element_type=F32) + b_r[...]
        )

    return pl.pallas_call(
        body,
        grid=(g,),
        in_specs=[
            pl.BlockSpec((blk, CH), lambda i: (i, 0)),
            pl.BlockSpec((CH, FEAT), lambda i: (0, 0)),
            pl.BlockSpec((1, FEAT), lambda i: (0, 0)),
        ],
        out_specs=pl.BlockSpec((blk, FEAT), lambda i: (i, 0)),
        out_shape=jax.ShapeDtypeStruct((n, FEAT), F32),
    )(inp, w, b2d)


RBLK = 1000
NSTEP = NUM_MASHUP // RBLK


def _rsqrtz(x):
    return jnp.where(x > 0, lax.rsqrt(x), 0.0)


def _gcn1(aw, v_m, v_s):
    def body(a_r, vm_r, vs_r, w_r, dega_r, accu_r, accda_r):
        i = pl.program_id(0)
        awb = a_r[...]
        a_lo = (awb & 1).astype(BF16)
        a_hi = (awb >> 16).astype(BF16)
        degm = lax.dot_general(
            a_lo + a_hi, jnp.ones((WPR, 8), BF16), (((1,), (0,)), ((), ())),
            preferred_element_type=F32,
        )[:, 0:1]
        dism = _rsqrtz(degm)
        x = (dism * vm_r[...]).astype(BF16)
        contrib_u = jnp.concatenate([
            lax.dot_general(a_lo, x, (((0,), (0,)), ((), ())),
                            preferred_element_type=F32),
            lax.dot_general(a_hi, x, (((0,), (0,)), ((), ())),
                            preferred_element_type=F32),
        ], axis=0)
        ones_r = jnp.ones((RBLK, 8), BF16)
        contrib_da = jnp.concatenate([
            lax.dot_general(a_lo, ones_r, (((0,), (0,)), ((), ())),
                            preferred_element_type=F32),
            lax.dot_general(a_hi, ones_r, (((0,), (0,)), ((), ())),
                            preferred_element_type=F32),
        ], axis=0)

        @pl.when(i == 0)
        def _():
            accu_r[...] = contrib_u
            accda_r[...] = contrib_da

        @pl.when(i > 0)
        def _():
            accu_r[...] += contrib_u
            accda_r[...] += contrib_da

        @pl.when(i == NSTEP - 1)
        def _():
            dega = accda_r[...]
            disa = _rsqrtz(dega[:, 0:1])
            w_r[...] = vs_r[...] + disa * accu_r[...]
            dega_r[...] = dega

    return pl.pallas_call(
        body,
        grid=(NSTEP,),
        in_specs=[
            pl.BlockSpec((RBLK, WPR), lambda i: (i, 0)),
            pl.BlockSpec((RBLK, FEAT), lambda i: (i, 0)),
            pl.BlockSpec((NUM_API, FEAT), lambda i: (0, 0)),
        ],
        out_specs=[
            pl.BlockSpec((NUM_API, FEAT), lambda i: (0, 0)),
            pl.BlockSpec((NUM_API, 8), lambda i: (0, 0)),
        ],
        out_shape=[
            jax.ShapeDtypeStruct((NUM_API, FEAT), F32),
            jax.ShapeDtypeStruct((NUM_API, 8), F32),
        ],
        scratch_shapes=[
            pltpu.VMEM((NUM_API, FEAT), F32),
            pltpu.VMEM((NUM_API, 8), F32),
        ],
    )(aw, v_m, v_s)


def _gcn2(aw, w, dega, z_m):
    b = z_m.shape[0]

    def body(a_r, w_r, dega_r, zm_r, pred_r, acc2_r):
        i = pl.program_id(0)
        awb = a_r[...]
        a_lo = (awb & 1).astype(BF16)
        a_hi = (awb >> 16).astype(BF16)
        disa = _rsqrtz(dega_r[:, 0:1])
        wa = (disa * w_r[...]).astype(BF16)
        traw = lax.dot_general(
            a_lo, wa[:WPR], (((1,), (0,)), ((), ())),
            preferred_element_type=F32,
        ) + lax.dot_general(
            a_hi, wa[WPR:], (((1,), (0,)), ((), ())),
            preferred_element_type=F32,
        )
        degm = lax.dot_general(
            a_lo + a_hi, jnp.ones((WPR, 8), BF16), (((1,), (0,)), ((), ())),
            preferred_element_type=F32,
        )[:, 0:1]
        dm2 = jnp.where(degm > 0, 1.0 / degm, 0.0)
        y = (dm2 * traw).astype(BF16)
        contrib = jnp.concatenate([
            lax.dot_general(a_lo, y, (((0,), (0,)), ((), ())),
                            preferred_element_type=F32),
            lax.dot_general(a_hi, y, (((0,), (0,)), ((), ())),
                            preferred_element_type=F32),
        ], axis=0)

        @pl.when(i == 0)
        def _():
            acc2_r[...] = contrib

        @pl.when(i > 0)
        def _():
            acc2_r[...] += contrib

        @pl.when(i == NSTEP - 1)
        def _():
            out_s = 0.25 * (w_r[...] + disa * acc2_r[...])
            pred_r[...] = lax.dot_general(
                zm_r[...], out_s, (((1,), (1,)), ((), ())),
                preferred_element_type=F32,
            )

    return pl.pallas_call(
        body,
        grid=(NSTEP,),
        in_specs=[
            pl.BlockSpec((RBLK, WPR), lambda i: (i, 0)),
            pl.BlockSpec((NUM_API, FEAT), lambda i: (0, 0)),
            pl.BlockSpec((NUM_API, 8), lambda i: (0, 0)),
            pl.BlockSpec((b, FEAT), lambda i: (0, 0)),
        ],
        out_specs=pl.BlockSpec((b, NUM_API), lambda i: (0, 0)),
        out_shape=jax.ShapeDtypeStruct((b, NUM_API), F32),
        scratch_shapes=[pltpu.VMEM((NUM_API, FEAT), F32)],
    )(aw, w, dega, z_m)


def kernel(x, mashup_embed, api_embed, domain_embed, edge_index,
           W_sde, b_sde, W_val, b_val, W_key, b_key, W_sie, b_sie):
    e = edge_index.shape[1]
    e2 = e // 2
    src_half = edge_index[0]
    dst_half = edge_index[1]

    a_words = _build_adjacency_words(src_half, dst_half, e2)
    aw = a_words.reshape(NUM_MASHUP, WPR)

    b_sde2 = b_sde.reshape(1, FEAT)
    b_val2 = b_val.reshape(1, FEAT)
    b_key2 = b_key.reshape(1, FEAT)
    b_sie2 = b_sie.reshape(1, FEAT)

    z_m = _attention(x, domain_embed, W_sde, b_sde2, W_val, b_val2, W_key, b_key2)
    v_m = _proj(mashup_embed, W_sde, b_sde2, 1000)
    v_s = _proj(api_embed, W_sie, b_sie2, 1000)

    w, dega = _gcn1(aw, v_m, v_s)
    pred = _gcn2(aw, w, dega, z_m)
    return pred
